# baseline (speedup 1.0000x reference)
import functools

import jax
import jax.numpy as jnp
from jax import lax
from jax.experimental import pallas as pl
from jax.experimental.pallas import tpu as pltpu
from jax.experimental.pallas import tpu_sc as plsc

VOCAB = 100000
EMBED_DIM = 16
BATCH = 16384

_NC = 2
_NS = 16
_NW = _NC * _NS
_LANES = 16

_CHUNK = 128
_B_PER_W = BATCH // _NW
_N_CHUNKS = _B_PER_W // _CHUNK


def _lookup_kernel(idx_hbm, table_hbm, out_hbm, idx_v, rows_v, sem):
    wid = lax.axis_index("s") * _NC + lax.axis_index("c")
    base = wid * _B_PER_W

    pltpu.sync_copy(idx_hbm.at[pl.ds(base, _B_PER_W)], idx_v)

    def remap(i, carry):
        sl = pl.ds(i * _LANES, _LANES)
        v = idx_v[sl]
        idx_v[sl] = jnp.where((v >= 0) & (v < VOCAB), v + 1, 0)
        return carry

    lax.fori_loop(0, _B_PER_W // _LANES, remap, 0)

    def fire(j, carry):
        sl = pl.ds(j * _CHUNK, _CHUNK)
        pltpu.async_copy(table_hbm.at[idx_v.at[sl]], rows_v.at[sl], sem)
        return carry

    lax.fori_loop(0, _N_CHUNKS, fire, 0)

    pltpu.make_async_copy(table_hbm.at[pl.ds(0, _B_PER_W)], rows_v, sem).wait()

    pltpu.sync_copy(rows_v, out_hbm.at[pl.ds(base, _B_PER_W)])


def kernel(user, table):
    mesh = plsc.VectorSubcoreMesh(core_axis_name="c", subcore_axis_name="s")
    run = functools.partial(
        pl.kernel,
        mesh=mesh,
        compiler_params=pltpu.CompilerParams(use_tc_tiling_on_sc=False),
        out_type=jax.ShapeDtypeStruct((BATCH, EMBED_DIM), jnp.float32),
        scratch_types=[
            pltpu.VMEM((_B_PER_W,), jnp.int32),
            pltpu.VMEM((_B_PER_W, EMBED_DIM), jnp.float32),
            pltpu.SemaphoreType.DMA,
        ],
    )(_lookup_kernel)
    return run(user.astype(jnp.int32), table)
